# baseline (speedup 1.0000x reference)
import functools

import jax
import jax.numpy as jnp
from jax import lax
from jax.experimental import pallas as pl
from jax.experimental.pallas import tpu as pltpu

_BS = 16384
_VPAD = 1024
_NBUF = 2


@functools.lru_cache(maxsize=None)
def _build(B, V, D):
    nblk = B // _BS
    assert nblk % _NBUF == 0

    def body(idx_hbm, tabt_ref, out_hbm, idxv, rows, isem, osem):
        def idx_fetch(blk, p):
            pltpu.async_copy(idx_hbm.at[blk], idxv.at[p], isem.at[p])

        for p in range(_NBUF):
            idx_fetch(p, p)

        def step(h, carry):
            for p in range(_NBUF):
                blk = h * _NBUF + p
                pltpu.make_async_copy(
                    idx_hbm.at[0], idxv.at[p], isem.at[p]).wait()

                @pl.when(blk >= _NBUF)
                def _():
                    pltpu.make_async_copy(
                        rows.at[p], out_hbm.at[pl.ds(0, _BS)], osem.at[p]
                    ).wait()

                idx16 = idxv[p, 0, :].astype(jnp.int16)
                io = lax.broadcasted_iota(jnp.int16, (_VPAD, _BS), 0)
                oh = jnp.where(io == idx16[None, :],
                               jnp.bfloat16(1), jnp.bfloat16(0))
                res = jnp.dot(tabt_ref[...], oh,
                              preferred_element_type=jnp.float32)
                rows[p] = res.T

                off = pl.multiple_of(blk * _BS, _BS)
                pltpu.async_copy(rows.at[p], out_hbm.at[pl.ds(off, _BS)],
                                 osem.at[p])

                @pl.when(blk + _NBUF < nblk)
                def _():
                    idx_fetch(blk + _NBUF, p)

            return carry

        lax.fori_loop(0, nblk // _NBUF, step, 0)

        for p in range(_NBUF):
            pltpu.make_async_copy(
                rows.at[p], out_hbm.at[pl.ds(0, _BS)], osem.at[p]).wait()

    return pl.pallas_call(
        body,
        in_specs=[
            pl.BlockSpec(memory_space=pltpu.MemorySpace.HBM),
            pl.BlockSpec(memory_space=pltpu.MemorySpace.VMEM),
        ],
        out_specs=pl.BlockSpec(memory_space=pltpu.MemorySpace.HBM),
        out_shape=jax.ShapeDtypeStruct((B, D), jnp.float32),
        scratch_shapes=[
            pltpu.VMEM((_NBUF, 1, _BS), jnp.int32),
            pltpu.VMEM((_NBUF, _BS, D), jnp.float32),
            pltpu.SemaphoreType.DMA((_NBUF,)),
            pltpu.SemaphoreType.DMA((_NBUF,)),
        ],
    )


def kernel(visit_order, pos_embed):
    R, S = visit_order.shape
    V, D = pos_embed.shape
    B = R * S
    idx = visit_order.reshape(B // _BS, 1, _BS).astype(jnp.int32)
    tabt = jnp.pad(pos_embed, ((0, _VPAD - V), (0, 0))).astype(jnp.bfloat16).T
    out = _build(B, V, D)(idx, tabt)
    return out.reshape(R, S, D)
